# baseline (speedup 1.0000x reference)
import functools

import jax
import jax.numpy as jnp
from jax import lax
from jax.experimental import pallas as pl
from jax.experimental.pallas import tpu as pltpu
from jax.experimental.pallas import tpu_sc as plsc

Q = 1024
D = 128
K = 100000
KB = 2048
KPAD = 100352
NRET = 8
SEGW = 16
NSEG = KPAD // SEGW
SEGB = KB // SEGW
NSEL = 16
QB = 256
OBSF = 3 * 16 * 16

NC = 2
NS = 16
L = 16
NW = NC * NS
QW = Q // NW
BQ = 1
NB = QW // BQ


def _dist_body(q_ref, k_ref, d2_ref, smin_ref, qsq_ref):
    i = pl.program_id(0)

    @pl.when(i == 0)
    def _():
        qs = jnp.sum(q_ref[...] * q_ref[...], axis=1)
        qsq_ref[...] = qs[None, :]

    q = q_ref[...]
    kblk = k_ref[...]
    ksq = jnp.sum(kblk * kblk, axis=1)
    qk = lax.dot_general(q, kblk, (((1,), (1,)), ((), ())),
                         preferred_element_type=jnp.float32)
    qsq = jnp.sum(q * q, axis=1, keepdims=True)
    d2_ref[...] = jnp.maximum(qsq + ksq[None, :] - 2.0 * qk, 0.0)
    qkt = lax.dot_general(kblk, q, (((1,), (1,)), ((), ())),
                          preferred_element_type=jnp.float32)
    d2t = jnp.maximum(qsq_ref[...] + ksq[:, None] - 2.0 * qkt, 0.0)
    smin_ref[...] = jnp.min(d2t.reshape(SEGB, SEGW, Q), axis=1)


def _distances(q, k):
    kp = jnp.pad(k, ((0, KPAD - K), (0, 0)), constant_values=1e4)
    grid = (KPAD // KB,)
    return pl.pallas_call(
        _dist_body,
        grid=grid,
        in_specs=[
            pl.BlockSpec((Q, D), lambda i: (0, 0)),
            pl.BlockSpec((KB, D), lambda i: (i, 0)),
        ],
        out_specs=[
            pl.BlockSpec((Q, KB), lambda i: (0, i)),
            pl.BlockSpec((SEGB, Q), lambda i: (i, 0)),
        ],
        out_shape=[
            jax.ShapeDtypeStruct((Q, KPAD), jnp.float32),
            jax.ShapeDtypeStruct((NSEG, Q), jnp.float32),
        ],
        scratch_shapes=[pltpu.VMEM((1, Q), jnp.float32)],
    )(q, kp)


def _segsel_body(smin_ref, ids_ref):
    s = smin_ref[...]
    iota = lax.broadcasted_iota(jnp.int32, s.shape, 0)
    ids = []
    for _ in range(NSEL):
        m = jnp.min(s, axis=0, keepdims=True)
        am = jnp.min(jnp.where(s == m, iota, jnp.int32(2**30)),
                     axis=0, keepdims=True)
        ids.append(am[0, :])
        s = jnp.where(iota == am, jnp.float32(jnp.inf), s)
    ids = jnp.stack(ids, axis=0)
    rank = jnp.sum((ids[None, :, :] < ids[:, None, :]).astype(jnp.int32),
                   axis=1)
    slot = lax.broadcasted_iota(jnp.int32, (NSEL, NSEL, QB), 0)
    placed = jnp.where(rank[None, :, :] == slot, ids[None, :, :], 0)
    ids_ref[...] = jnp.transpose(jnp.sum(placed, axis=1), (1, 0))


def _select_segments(smin):
    return pl.pallas_call(
        _segsel_body,
        grid=(Q // QB,),
        in_specs=[pl.BlockSpec((NSEG, QB), lambda i: (0, i))],
        out_specs=pl.BlockSpec((QB, NSEL), lambda i: (i, 0)),
        out_shape=jax.ShapeDtypeStruct((Q, NSEL), jnp.int32),
    )(smin)


def _top8_body(cand_ref, cols_ref, idx_ref):
    s = cand_ref[...]
    cols = cols_ref[...]
    iota = lax.broadcasted_iota(jnp.int32, s.shape, 1)
    for r in range(NRET):
        m = jnp.min(s, axis=1, keepdims=True)
        am = jnp.min(jnp.where(s == m, iota, jnp.int32(2**30)),
                     axis=1, keepdims=True)
        hit = iota == am
        idx_ref[:, r] = jnp.sum(jnp.where(hit, cols, 0), axis=1)
        s = jnp.where(hit, jnp.float32(jnp.inf), s)


def _top8(cand, cols):
    return pl.pallas_call(
        _top8_body,
        grid=(Q // QB,),
        in_specs=[
            pl.BlockSpec((QB, NSEL * SEGW), lambda i: (i, 0)),
            pl.BlockSpec((QB, NSEL * SEGW), lambda i: (i, 0)),
        ],
        out_specs=pl.BlockSpec((QB, NRET), lambda i: (i, 0)),
        out_shape=jax.ShapeDtypeStruct((Q, NRET), jnp.int32),
    )(cand, cols)


def _vgather(x, idx):
    dnums = lax.GatherDimensionNumbers(
        offset_dims=(), collapsed_slice_dims=(0,), start_index_map=(0,))
    return lax.gather(x, idx[:, None], dnums, (1,),
                      mode=lax.GatherScatterMode.PROMISE_IN_BOUNDS)


def _allmin(x, iota):
    for s in (8, 4, 2, 1):
        x = jnp.minimum(x, _vgather(x, iota ^ s))
    return x

def _sc_body(d2v, segids_hbm, obsf, outf,
             segid_l, candidx, candbuf, idxsel, obsbuf0, obsbuf1,
             sem0, sem1):
    wid = lax.axis_index("s") * NC + lax.axis_index("c")
    base = wid * QW
    iota = lax.broadcasted_iota(jnp.int32, (L,), 0)

    pltpu.sync_copy(segids_hbm, segid_l)

    def batch_body(b, carry):
        q0 = b * BQ
        segvecs = []
        for u in range(BQ):
            qglob = base + q0 + u
            segvec = segid_l[pl.ds(qglob * NSEL, NSEL)]
            segvecs.append(segvec)
            candidx[pl.ds(u * L, L)] = (
                lax.shift_right_logical(segvec, 3) + qglob * (KPAD // 128))
        pltpu.async_copy(d2v.at[candidx], candbuf, sem0).wait()
        for u in range(BQ):
            qi = q0 + u
            vals = []
            for g in range(NSEL):
                off = (_vgather(segvecs[u],
                                jnp.full((L,), g, jnp.int32)) & 7)
                v = jnp.zeros((L,), jnp.float32)
                for c in range(8):
                    ind = jnp.where(off == c, jnp.float32(1.0),
                                    jnp.float32(0.0))
                    v = v + candbuf[u * NSEL + g, pl.ds(c * L, L)] * ind
                vals.append(v)
            flats = [jnp.full((L,), g * L, jnp.int32) + iota
                     for g in range(NSEL)]
            possel = jnp.zeros((L,), jnp.int32)
            for r in range(NRET):
                m = functools.reduce(jnp.minimum, vals)
                ms = _allmin(m, iota)
                p = jnp.full((L,), jnp.int32(2**30))
                for g in range(NSEL):
                    p = jnp.minimum(
                        p, jnp.where(vals[g] == ms, flats[g],
                                     jnp.int32(2**30)))
                ps = _allmin(p, iota)
                for g in range(NSEL):
                    vals[g] = jnp.where(
                        (vals[g] == ms) & (flats[g] == ps),
                        jnp.float32(jnp.inf), vals[g])
                possel = jnp.where(iota == r, ps, possel)
            srank = lax.shift_right_logical(possel, 4)
            t = possel & (SEGW - 1)
            gsel = _vgather(segvecs[u], srank)
            obsrow = gsel * SEGW + t
            idxsel[pl.ds(qi * L, L)] = obsrow
        return carry

    lax.fori_loop(0, NB, batch_body, 0)

    bufs = (obsbuf0, obsbuf1)
    sems = (sem0, sem1)

    def write_out(qi, buf):
        for j in range(NRET):
            pltpu.sync_copy(buf.at[j], outf.at[j * Q + base + qi])

    prev = pltpu.async_copy(obsf.at[idxsel.at[pl.ds(0, NRET)]], bufs[0],
                            sems[0])
    for qi in range(QW):
        if qi + 1 < QW:
            nxt = pltpu.async_copy(
                obsf.at[idxsel.at[pl.ds((qi + 1) * L, NRET)]],
                bufs[(qi + 1) % 2], sems[(qi + 1) % 2])
        prev.wait()
        write_out(qi, bufs[qi % 2])
        if qi + 1 < QW:
            prev = nxt


def _sc_select_gather(d2, segids, obs):
    d2v = d2.reshape(Q * (KPAD // 128), 128)
    segf = segids.reshape(Q * NSEL)
    obsf = obs.reshape(K, OBSF)
    mesh = plsc.VectorSubcoreMesh(core_axis_name="c", subcore_axis_name="s")
    run = pl.kernel(
        _sc_body,
        out_type=jax.ShapeDtypeStruct((NRET * Q, OBSF), jnp.float32),
        mesh=mesh,
        scratch_types=[
            pltpu.VMEM((Q * NSEL,), jnp.int32),
            pltpu.VMEM((BQ * L,), jnp.int32),
            pltpu.VMEM((BQ * NSEL, 128), jnp.float32),
            pltpu.VMEM((QW * L,), jnp.int32),
            pltpu.VMEM((NRET, OBSF), jnp.float32),
            pltpu.VMEM((NRET, OBSF), jnp.float32),
            pltpu.SemaphoreType.DMA,
            pltpu.SemaphoreType.DMA,
        ],
    )
    return run(d2v, segf, obsf)


def kernel(q, k, obs):
    d2, smin = _distances(q, k)
    segids = _select_segments(smin)
    cols = (segids[:, :, None] * SEGW
            + jnp.arange(SEGW, dtype=jnp.int32)[None, None, :]
            ).reshape(Q, NSEL * SEGW)
    cand = jnp.take_along_axis(d2, cols, axis=1)
    idx = _top8(cand, cols)
    sel = obs[idx]
    return jnp.transpose(sel, (1, 0, 2, 3, 4))


# speedup vs baseline: 1.9294x; 1.9294x over previous
import functools

import jax
import jax.numpy as jnp
from jax import lax
from jax.experimental import pallas as pl
from jax.experimental.pallas import tpu as pltpu
from jax.experimental.pallas import tpu_sc as plsc

Q = 1024
D = 128
K = 100000
KB = 2048
KPAD = 100352
NRET = 8
SEGW = 16
NSEG = KPAD // SEGW
SEGB = KB // SEGW
NSEL = 16
QB = 256
OBSF = 3 * 16 * 16

NC = 2
NS = 16
L = 16
NW = NC * NS
QW = Q // NW
BQ = 1
NB = QW // BQ


def _dist_body(q_ref, k_ref, d2_ref, smin_ref, qsq_ref):
    i = pl.program_id(0)

    @pl.when(i == 0)
    def _():
        qs = jnp.sum(q_ref[...] * q_ref[...], axis=1)
        qsq_ref[...] = qs[None, :]

    q = q_ref[...]
    kblk = k_ref[...]
    ksq = jnp.sum(kblk * kblk, axis=1)
    qk = lax.dot_general(q, kblk, (((1,), (1,)), ((), ())),
                         preferred_element_type=jnp.float32)
    qsq = jnp.sum(q * q, axis=1, keepdims=True)
    d2_ref[...] = jnp.maximum(qsq + ksq[None, :] - 2.0 * qk, 0.0)
    qkt = lax.dot_general(kblk, q, (((1,), (1,)), ((), ())),
                          preferred_element_type=jnp.float32)
    d2t = jnp.maximum(qsq_ref[...] + ksq[:, None] - 2.0 * qkt, 0.0)
    smin_ref[...] = jnp.min(d2t.reshape(SEGB, SEGW, Q), axis=1)


def _distances(q, k):
    kp = jnp.pad(k, ((0, KPAD - K), (0, 0)), constant_values=1e4)
    grid = (KPAD // KB,)
    return pl.pallas_call(
        _dist_body,
        grid=grid,
        in_specs=[
            pl.BlockSpec((Q, D), lambda i: (0, 0)),
            pl.BlockSpec((KB, D), lambda i: (i, 0)),
        ],
        out_specs=[
            pl.BlockSpec((Q, KB), lambda i: (0, i)),
            pl.BlockSpec((SEGB, Q), lambda i: (i, 0)),
        ],
        out_shape=[
            jax.ShapeDtypeStruct((Q, KPAD), jnp.float32),
            jax.ShapeDtypeStruct((NSEG, Q), jnp.float32),
        ],
        scratch_shapes=[pltpu.VMEM((1, Q), jnp.float32)],
    )(q, kp)


def _segsel_body(smin_ref, ids_ref):
    s = smin_ref[...]
    iota = lax.broadcasted_iota(jnp.int32, s.shape, 0)
    ids = []
    for _ in range(NSEL):
        m = jnp.min(s, axis=0, keepdims=True)
        am = jnp.min(jnp.where(s == m, iota, jnp.int32(2**30)),
                     axis=0, keepdims=True)
        ids.append(am[0, :])
        s = jnp.where(iota == am, jnp.float32(jnp.inf), s)
    ids = jnp.stack(ids, axis=0)
    rank = jnp.sum((ids[None, :, :] < ids[:, None, :]).astype(jnp.int32),
                   axis=1)
    slot = lax.broadcasted_iota(jnp.int32, (NSEL, NSEL, QB), 0)
    placed = jnp.where(rank[None, :, :] == slot, ids[None, :, :], 0)
    ids_ref[...] = jnp.transpose(jnp.sum(placed, axis=1), (1, 0))


def _select_segments(smin):
    return pl.pallas_call(
        _segsel_body,
        grid=(Q // QB,),
        in_specs=[pl.BlockSpec((NSEG, QB), lambda i: (0, i))],
        out_specs=pl.BlockSpec((QB, NSEL), lambda i: (i, 0)),
        out_shape=jax.ShapeDtypeStruct((Q, NSEL), jnp.int32),
    )(smin)


def _top8_body(cand_ref, cols_ref, idx_ref):
    s = cand_ref[...]
    cols = cols_ref[...]
    iota = lax.broadcasted_iota(jnp.int32, s.shape, 1)
    for r in range(NRET):
        m = jnp.min(s, axis=1, keepdims=True)
        am = jnp.min(jnp.where(s == m, iota, jnp.int32(2**30)),
                     axis=1, keepdims=True)
        hit = iota == am
        idx_ref[:, r] = jnp.sum(jnp.where(hit, cols, 0), axis=1)
        s = jnp.where(hit, jnp.float32(jnp.inf), s)


def _top8(cand, cols):
    return pl.pallas_call(
        _top8_body,
        grid=(Q // QB,),
        in_specs=[
            pl.BlockSpec((QB, NSEL * SEGW), lambda i: (i, 0)),
            pl.BlockSpec((QB, NSEL * SEGW), lambda i: (i, 0)),
        ],
        out_specs=pl.BlockSpec((QB, NRET), lambda i: (i, 0)),
        out_shape=jax.ShapeDtypeStruct((Q, NRET), jnp.int32),
    )(cand, cols)


def _vgather(x, idx):
    dnums = lax.GatherDimensionNumbers(
        offset_dims=(), collapsed_slice_dims=(0,), start_index_map=(0,))
    return lax.gather(x, idx[:, None], dnums, (1,),
                      mode=lax.GatherScatterMode.PROMISE_IN_BOUNDS)


def _allmin(x, iota):
    for s in (8, 4, 2, 1):
        x = jnp.minimum(x, _vgather(x, iota ^ s))
    return x

def _sc_body(d2v, segids_hbm, obsf, outf,
             segid_l, candidx, candbuf, idxsel, obsbuf0, obsbuf1,
             sem0, sem1):
    wid = lax.axis_index("s") * NC + lax.axis_index("c")
    base = wid * QW
    iota = lax.broadcasted_iota(jnp.int32, (L,), 0)

    pltpu.sync_copy(segids_hbm, segid_l)

    def batch_body(b, carry):
        q0 = b * BQ
        segvecs = []
        for u in range(BQ):
            qglob = base + q0 + u
            segvec = segid_l[pl.ds(qglob * NSEL, NSEL)]
            segvecs.append(segvec)
            candidx[pl.ds(u * L, L)] = (
                lax.shift_right_logical(segvec, 3) + qglob * (KPAD // 128))
        pltpu.async_copy(d2v.at[candidx], candbuf, sem0).wait()
        for u in range(BQ):
            qi = q0 + u
            vals = []
            for g in range(NSEL):
                off = (_vgather(segvecs[u],
                                jnp.full((L,), g, jnp.int32)) & 7)
                v = jnp.zeros((L,), jnp.float32)
                for c in range(8):
                    ind = jnp.where(off == c, jnp.float32(1.0),
                                    jnp.float32(0.0))
                    v = v + candbuf[u * NSEL + g, pl.ds(c * L, L)] * ind
                vals.append(v)
            flats = [jnp.full((L,), g * L, jnp.int32) + iota
                     for g in range(NSEL)]
            possel = jnp.zeros((L,), jnp.int32)
            for r in range(NRET):
                m = functools.reduce(jnp.minimum, vals)
                ms = _allmin(m, iota)
                p = jnp.full((L,), jnp.int32(2**30))
                for g in range(NSEL):
                    p = jnp.minimum(
                        p, jnp.where(vals[g] == ms, flats[g],
                                     jnp.int32(2**30)))
                ps = _allmin(p, iota)
                for g in range(NSEL):
                    vals[g] = jnp.where(
                        (vals[g] == ms) & (flats[g] == ps),
                        jnp.float32(jnp.inf), vals[g])
                possel = jnp.where(iota == r, ps, possel)
            srank = lax.shift_right_logical(possel, 4)
            t = possel & (SEGW - 1)
            gsel = _vgather(segvecs[u], srank)
            obsrow = gsel * SEGW + t
            idxsel[pl.ds(qi * L, L)] = obsrow
        return carry

    lax.fori_loop(0, NB, batch_body, 0)

    bufs = (obsbuf0, obsbuf1)
    sems = (sem0, sem1)

    def write_out(qi, buf):
        for j in range(NRET):
            pltpu.sync_copy(buf.at[j], outf.at[j * Q + base + qi])

    prev = pltpu.async_copy(obsf.at[idxsel.at[pl.ds(0, NRET)]], bufs[0],
                            sems[0])
    for qi in range(QW):
        if qi + 1 < QW:
            nxt = pltpu.async_copy(
                obsf.at[idxsel.at[pl.ds((qi + 1) * L, NRET)]],
                bufs[(qi + 1) % 2], sems[(qi + 1) % 2])
        prev.wait()
        write_out(qi, bufs[qi % 2])
        if qi + 1 < QW:
            prev = nxt


def _sc_select_gather(d2, segids, obs):
    d2v = d2.reshape(Q * (KPAD // 128), 128)
    segf = segids.reshape(Q * NSEL)
    obsf = obs.reshape(K, OBSF)
    mesh = plsc.VectorSubcoreMesh(core_axis_name="c", subcore_axis_name="s")
    run = pl.kernel(
        _sc_body,
        out_type=jax.ShapeDtypeStruct((NRET * Q, OBSF), jnp.float32),
        mesh=mesh,
        scratch_types=[
            pltpu.VMEM((Q * NSEL,), jnp.int32),
            pltpu.VMEM((BQ * L,), jnp.int32),
            pltpu.VMEM((BQ * NSEL, 128), jnp.float32),
            pltpu.VMEM((QW * L,), jnp.int32),
            pltpu.VMEM((NRET, OBSF), jnp.float32),
            pltpu.VMEM((NRET, OBSF), jnp.float32),
            pltpu.SemaphoreType.DMA,
            pltpu.SemaphoreType.DMA,
        ],
    )
    return run(d2v, segf, obsf)


GB = 64
RPW = NRET * Q // NW


def _scg_body(idx_hbm, obsf, outf, idx_v, buf0, buf1, sem0, sem1):
    wid = lax.axis_index("s") * NC + lax.axis_index("c")
    base = wid * RPW
    pltpu.sync_copy(idx_hbm.at[pl.ds(base, RPW)], idx_v)
    bufs = (buf0, buf1)
    sems = (sem0, sem1)
    nchunk = RPW // GB
    prev = pltpu.async_copy(obsf.at[idx_v.at[pl.ds(0, GB)]], bufs[0],
                            sems[0])
    for c in range(nchunk):
        if c + 1 < nchunk:
            nxt = pltpu.async_copy(
                obsf.at[idx_v.at[pl.ds((c + 1) * GB, GB)]],
                bufs[(c + 1) % 2], sems[(c + 1) % 2])
        prev.wait()
        pltpu.sync_copy(bufs[c % 2], outf.at[pl.ds(base + c * GB, GB)])
        if c + 1 < nchunk:
            prev = nxt


def _sc_obs_gather(idxt, obs):
    obsf = obs.reshape(K, OBSF)
    mesh = plsc.VectorSubcoreMesh(core_axis_name="c", subcore_axis_name="s")
    run = pl.kernel(
        _scg_body,
        out_type=jax.ShapeDtypeStruct((NRET * Q, OBSF), jnp.float32),
        mesh=mesh,
        scratch_types=[
            pltpu.VMEM((RPW,), jnp.int32),
            pltpu.VMEM((GB, OBSF), jnp.float32),
            pltpu.VMEM((GB, OBSF), jnp.float32),
            pltpu.SemaphoreType.DMA,
            pltpu.SemaphoreType.DMA,
        ],
    )
    return run(idxt, obsf)


def kernel(q, k, obs):
    d2, smin = _distances(q, k)
    segids = _select_segments(smin)
    seg8 = segids >> 3
    off = segids & 7
    grows = (jnp.arange(Q, dtype=jnp.int32)[:, None] * (KPAD // 128)
             + seg8).reshape(-1)
    cands8 = jnp.take(d2.reshape(Q * (KPAD // 128), 128), grows,
                      axis=0).reshape(Q, NSEL, 128)
    wincols = (off[:, :, None] * SEGW
               + jnp.arange(SEGW, dtype=jnp.int32)[None, None, :])
    cand = jnp.take_along_axis(cands8, wincols, axis=2).reshape(
        Q, NSEL * SEGW)
    cols = (segids[:, :, None] * SEGW
            + jnp.arange(SEGW, dtype=jnp.int32)[None, None, :]
            ).reshape(Q, NSEL * SEGW)
    idx = _top8(cand, cols)
    idxt = jnp.transpose(idx, (1, 0)).reshape(NRET * Q)
    outf = _sc_obs_gather(idxt, obs)
    return outf.reshape(NRET, Q, 3, 16, 16)
